# Initial kernel scaffold; baseline (speedup 1.0000x reference)
import jax
import jax.numpy as jnp
from jax.experimental import pallas as pl


def kernel(x, edge_index, batch, W1r, W1n, b1, p1, W2r, W2n, b2, p2, W3r, W3n, b3, p3, L1w, L1b, L2w, L2b, L3w, L3b):
    raise NotImplementedError("write your pallas kernel here")


# speedup vs baseline: 5.2740x; 5.2740x over previous
import functools

import jax
import jax.numpy as jnp
from jax import lax
from jax.experimental import pallas as pl
from jax.experimental.pallas import tpu as pltpu
from jax.experimental.pallas import tpu_sc as plsc

_N = 10000
_NP = 10240
_E = 160000
_EP = 163840
_NG = 64
_RATIO = 0.8
_HID = 128
_CH = 128
_NSUB = 16
_NW = 2 * _NSUB
_EPW = _EP // _NW
_CPT = _EPW // _CH
_RPT = _NP // _NSUB


def _sc_segsum(xn, src, dst):
  mesh = plsc.VectorSubcoreMesh(core_axis_name="c", subcore_axis_name="s")

  def body(xn_hbm, src_hbm, dst_hbm, out_hbm, src_v, dst_v, rows_v, agg_sh,
           sem):
    cid = lax.axis_index("c")
    sid = lax.axis_index("s")
    wid = cid * _NSUB + sid

    zero = jnp.zeros((16,), jnp.float32)

    def zbody(i, _):
      r = i // (_HID // 16)
      c = i % (_HID // 16)
      rows_v[r, pl.ds(c * 16, 16)] = zero
      return 0

    lax.fori_loop(0, _CH * (_HID // 16), zbody, 0)
    for j in range(_RPT // _CH):
      pltpu.sync_copy(rows_v, agg_sh.at[pl.ds(sid * _RPT + j * _CH, _CH)])
    plsc.subcore_barrier()

    def chunk(ci, _):
      base = wid * _EPW + ci * _CH
      pltpu.sync_copy(src_hbm.at[pl.ds(base, _CH)], src_v)
      pltpu.sync_copy(dst_hbm.at[pl.ds(base, _CH)], dst_v)
      pltpu.async_copy(xn_hbm.at[src_v], rows_v, sem).wait()
      pltpu.sync_copy(rows_v, agg_sh.at[dst_v], add=True)
      return 0

    lax.fori_loop(0, _CPT, chunk, 0)

    plsc.subcore_barrier()
    pltpu.sync_copy(agg_sh.at[pl.ds(sid * _RPT, _RPT)],
                    out_hbm.at[cid, pl.ds(sid * _RPT, _RPT)])

  return pl.kernel(
      body,
      out_type=jax.ShapeDtypeStruct((2, _NP, _HID), jnp.float32),
      mesh=mesh,
      scratch_types=[
          pltpu.VMEM((_CH,), jnp.int32),
          pltpu.VMEM((_CH,), jnp.int32),
          pltpu.VMEM((_CH, _HID), jnp.float32),
          pltpu.VMEM_SHARED((_NP, _HID), jnp.float32),
          pltpu.SemaphoreType.DMA,
      ],
  )(xn, src, dst)


def _mm2(h, Wr, Wn):
  K = h.shape[1]
  MB = 1024

  def body(h_ref, wr_ref, wn_ref, xr_ref, xn_ref):
    hb = h_ref[...]
    xr_ref[...] = jnp.dot(hb, wr_ref[...], preferred_element_type=jnp.float32)
    xn_ref[...] = jnp.dot(hb, wn_ref[...], preferred_element_type=jnp.float32)

  return pl.pallas_call(
      body,
      grid=(_NP // MB,),
      in_specs=[
          pl.BlockSpec((MB, K), lambda i: (i, 0)),
          pl.BlockSpec((K, _HID), lambda i: (0, 0)),
          pl.BlockSpec((K, _HID), lambda i: (0, 0)),
      ],
      out_specs=[
          pl.BlockSpec((MB, _HID), lambda i: (i, 0)),
          pl.BlockSpec((MB, _HID), lambda i: (i, 0)),
      ],
      out_shape=[
          jax.ShapeDtypeStruct((_NP, _HID), jnp.float32),
          jax.ShapeDtypeStruct((_NP, _HID), jnp.float32),
      ],
  )(h, Wr, Wn)


def _score(xr, a0, a1, b, p):
  MB = 1024

  def body(xr_ref, a0_ref, a1_ref, b_ref, p_ref, h_ref, s_ref):
    pv = p_ref[...]
    nrm = jnp.sqrt(jnp.sum(pv * pv)) + 1e-16
    h = jnp.maximum(xr_ref[...] + a0_ref[...] + a1_ref[...] + b_ref[...], 0.0)
    h_ref[...] = h
    sv = jnp.sum(h * pv, axis=1, keepdims=True) / nrm
    s_ref[...] = jnp.tanh(sv)

  return pl.pallas_call(
      body,
      grid=(_NP // MB,),
      in_specs=[
          pl.BlockSpec((MB, _HID), lambda i: (i, 0)),
          pl.BlockSpec((MB, _HID), lambda i: (i, 0)),
          pl.BlockSpec((MB, _HID), lambda i: (i, 0)),
          pl.BlockSpec((1, _HID), lambda i: (0, 0)),
          pl.BlockSpec((1, _HID), lambda i: (0, 0)),
      ],
      out_specs=[
          pl.BlockSpec((MB, _HID), lambda i: (i, 0)),
          pl.BlockSpec((MB, 1), lambda i: (i, 0)),
      ],
      out_shape=[
          jax.ShapeDtypeStruct((_NP, _HID), jnp.float32),
          jax.ShapeDtypeStruct((_NP, 1), jnp.float32),
      ],
  )(xr, a0, a1, b, p)


def _rank(s_c, bc_c, bo_c, t_c, s_r, bc_r, bo_r, t_r):
  MB = 1024
  NB = _NP // MB

  def body(si_ref, bci_ref, boi_ref, ti_ref, sj_ref, bcj_ref, boj_ref,
           tj_ref, rank_ref, cnt_ref, cntt_ref):
    i = pl.program_id(0)
    j = pl.program_id(1)

    @pl.when(j == 0)
    def _():
      rank_ref[...] = jnp.zeros_like(rank_ref)
      bci = bci_ref[...]
      gi = lax.broadcasted_iota(jnp.int32, (1, 128), 1)
      eq = (bci == gi).astype(jnp.int32)

      @pl.when(i == 0)
      def _():
        cnt_ref[...] = jnp.zeros_like(cnt_ref)

      cnt_ref[...] += jnp.sum(eq, axis=0, keepdims=True)

    @pl.when((i == 0) & (j == 0))
    def _():
      cntt_ref[...] = jnp.zeros_like(cntt_ref)

    @pl.when(i == 0)
    def _():
      bcj = bcj_ref[...]
      gc = lax.broadcasted_iota(jnp.int32, (128, 1), 0)
      eqt = (bcj == gc).astype(jnp.int32)
      cntt_ref[...] += jnp.sum(eqt, axis=1, keepdims=True)

    lo_i = boi_ref[0, 0]
    hi_i = boi_ref[MB - 1, 0]
    lo_j = boj_ref[0, 0]
    hi_j = boj_ref[0, MB - 1]

    @pl.when((lo_j <= hi_i) & (hi_j >= lo_i))
    def _():
      si = si_ref[...]
      ti = ti_ref[...]
      bi = bci_ref[...]
      sj = sj_ref[...]
      tj = tj_ref[...]
      bj = bcj_ref[...]
      better = (sj > si) | ((sj == si) & (tj < ti))
      m = (better & (bj == bi)).astype(jnp.int32)
      rank_ref[...] += jnp.sum(m, axis=1, keepdims=True)

  return pl.pallas_call(
      body,
      grid=(NB, NB),
      in_specs=[
          pl.BlockSpec((MB, 1), lambda i, j: (i, 0)),
          pl.BlockSpec((MB, 1), lambda i, j: (i, 0)),
          pl.BlockSpec((MB, 1), lambda i, j: (i, 0)),
          pl.BlockSpec((MB, 1), lambda i, j: (i, 0)),
          pl.BlockSpec((1, MB), lambda i, j: (0, j)),
          pl.BlockSpec((1, MB), lambda i, j: (0, j)),
          pl.BlockSpec((1, MB), lambda i, j: (0, j)),
          pl.BlockSpec((1, MB), lambda i, j: (0, j)),
      ],
      out_specs=[
          pl.BlockSpec((MB, 1), lambda i, j: (i, 0)),
          pl.BlockSpec((1, 128), lambda i, j: (0, 0)),
          pl.BlockSpec((128, 1), lambda i, j: (0, 0)),
      ],
      out_shape=[
          jax.ShapeDtypeStruct((_NP, 1), jnp.int32),
          jax.ShapeDtypeStruct((1, 128), jnp.int32),
          jax.ShapeDtypeStruct((128, 1), jnp.int32),
      ],
  )(s_c, bc_c, bo_c, t_c, s_r, bc_r, bo_r, t_r)


def _apply(h, s_c, rank_c, cnt, bc_c, bo_c):
  MB = 1024
  NB = _NP // MB

  def body(h_ref, s_ref, r_ref, cnt_ref, bc_ref, bo_ref, hn_ref, bn_ref,
           gmax_ref, gsum_ref):
    pid = pl.program_id(0)

    @pl.when(pid == 0)
    def _():
      gmax_ref[...] = jnp.full_like(gmax_ref[...], -jnp.inf)
      gsum_ref[...] = jnp.zeros_like(gsum_ref[...])

    cntv = cnt_ref[...]
    kv = jnp.ceil(cntv.astype(jnp.float32) * _RATIO).astype(jnp.int32)
    bc = bc_ref[...]
    gi = lax.broadcasted_iota(jnp.int32, (1, 128), 1)
    kn = jnp.sum(jnp.where(bc == gi, kv, 0), axis=1, keepdims=True)
    rk = r_ref[...]
    sv = s_ref[...]
    keep = (rk < kn) & (bc < _NG)
    hn = jnp.where(keep, h_ref[...] * sv, 0.0)
    hn_ref[...] = hn
    bn = jnp.where(keep, bc, _NG)
    bn_ref[...] = bn

    glo = bo_ref[0, 0]
    ghi = jnp.minimum(bo_ref[MB - 1, 0], _NG - 1)

    def gbody(g, _):
      m = bn == g
      vals = jnp.where(m, hn, -jnp.inf)
      cur = gmax_ref[pl.ds(g, 1), :]
      gmax_ref[pl.ds(g, 1), :] = jnp.maximum(
          cur, jnp.max(vals, axis=0, keepdims=True))
      gsum_ref[pl.ds(g, 1), :] += jnp.sum(
          jnp.where(m, hn, 0.0), axis=0, keepdims=True)
      return 0

    lax.fori_loop(glo, ghi + 1, gbody, 0)

  return pl.pallas_call(
      body,
      grid=(NB,),
      in_specs=[
          pl.BlockSpec((MB, _HID), lambda i: (i, 0)),
          pl.BlockSpec((MB, 1), lambda i: (i, 0)),
          pl.BlockSpec((MB, 1), lambda i: (i, 0)),
          pl.BlockSpec((1, 128), lambda i: (0, 0)),
          pl.BlockSpec((MB, 1), lambda i: (i, 0)),
          pl.BlockSpec((MB, 1), lambda i: (i, 0)),
      ],
      out_specs=[
          pl.BlockSpec((MB, _HID), lambda i: (i, 0)),
          pl.BlockSpec((MB, 1), lambda i: (i, 0)),
          pl.BlockSpec((_NG, 128), lambda i: (0, 0)),
          pl.BlockSpec((_NG, 128), lambda i: (0, 0)),
      ],
      out_shape=[
          jax.ShapeDtypeStruct((_NP, _HID), jnp.float32),
          jax.ShapeDtypeStruct((_NP, 1), jnp.int32),
          jax.ShapeDtypeStruct((_NG, 128), jnp.float32),
          jax.ShapeDtypeStruct((_NG, 128), jnp.float32),
      ],
  )(h, s_c, rank_c, cnt, bc_c, bo_c)


def _head(m1, s1, c1, m2, s2, c2, m3, s3, c3, w1a, w1b, b1, w2, b2, w3, b3):
  def body(m1_ref, s1_ref, c1_ref, m2_ref, s2_ref, c2_ref, m3_ref, s3_ref,
           c3_ref, w1a_ref, w1b_ref, b1_ref, w2_ref, b2_ref, w3_ref, b3_ref,
           out_ref):

    def rd(mref, sref, cref):
      cv = cref[...]
      kcol = jnp.ceil(cv.astype(jnp.float32) * _RATIO)[:_NG, :]
      mx = mref[...]
      mx = jnp.where(jnp.isfinite(mx), mx, 0.0)
      mean = sref[...] / jnp.maximum(kcol, 1.0)
      return mx, mean

    mA, eA = rd(m1_ref, s1_ref, c1_ref)
    mB, eB = rd(m2_ref, s2_ref, c2_ref)
    mC, eC = rd(m3_ref, s3_ref, c3_ref)
    zmax = mA + mB + mC
    zmean = eA + eB + eC
    hh = jnp.maximum(
        jnp.dot(zmax, w1a_ref[...], preferred_element_type=jnp.float32) +
        jnp.dot(zmean, w1b_ref[...], preferred_element_type=jnp.float32) +
        b1_ref[...], 0.0)
    hh = jnp.maximum(
        jnp.dot(hh, w2_ref[...], preferred_element_type=jnp.float32) +
        b2_ref[...], 0.0)
    o = jnp.dot(hh, w3_ref[...], preferred_element_type=jnp.float32) + b3_ref[...]
    o = o - jnp.max(o, axis=1, keepdims=True)
    out_ref[...] = o - jnp.log(jnp.sum(jnp.exp(o), axis=1, keepdims=True))

  return pl.pallas_call(
      body,
      out_shape=jax.ShapeDtypeStruct((_NG, 16), jnp.float32),
  )(m1, s1, c1, m2, s2, c2, m3, s3, c3, w1a, w1b, b1, w2, b2, w3, b3)


def kernel(x, edge_index, batch, W1r, W1n, b1, p1, W2r, W2n, b2, p2, W3r, W3n,
           b3, p3, L1w, L1b, L2w, L2b, L3w, L3b):
  xp = jnp.zeros((_NP, x.shape[1]), jnp.float32).at[:_N].set(x)
  pad_e = jnp.full((_EP - _E,), _N, jnp.int32)
  srcp = jnp.concatenate([edge_index[0], pad_e])
  dstp = jnp.concatenate([edge_index[1], pad_e])
  bor = jnp.full((_NP,), _NG, jnp.int32).at[:_N].set(batch)
  bor_c = bor.reshape(_NP, 1)
  bor_r = bor.reshape(1, _NP)
  bc_c, bc_r = bor_c, bor_r
  t_c = jnp.arange(_NP, dtype=jnp.int32).reshape(_NP, 1)
  t_r = t_c.reshape(1, _NP)
  hprev = xp

  reads = []
  for (Wr, Wn, bb, pp) in ((W1r, W1n, b1, p1), (W2r, W2n, b2, p2),
                           (W3r, W3n, b3, p3)):
    xr, xn = _mm2(hprev, Wr, Wn)
    part = _sc_segsum(xn, srcp, dstp)
    h, s_c = _score(xr, part[0], part[1], bb.reshape(1, _HID),
                    pp.reshape(1, _HID))
    s_r = s_c.reshape(1, _NP)
    rank_c, cnt, cntt = _rank(s_c, bc_c, bor_c, t_c, s_r, bc_r, bor_r, t_r)
    hn, bn_c, gmax, gsum = _apply(h, s_c, rank_c, cnt, bc_c, bor_c)
    reads.append((gmax, gsum, cntt))
    hprev = hn
    bc_c, bc_r = bn_c, bn_c.reshape(1, _NP)
    t_c, t_r = rank_c, rank_c.reshape(1, _NP)

  (m1, s1, c1), (m2, s2, c2), (m3, s3, c3) = reads
  return _head(m1, s1, c1, m2, s2, c2, m3, s3, c3, L1w[:_HID], L1w[_HID:],
               L1b.reshape(1, _HID), L2w, L2b.reshape(1, 64), L3w,
               L3b.reshape(1, 16))


# speedup vs baseline: 5.5765x; 1.0574x over previous
import functools

import jax
import jax.numpy as jnp
from jax import lax
from jax.experimental import pallas as pl
from jax.experimental.pallas import tpu as pltpu
from jax.experimental.pallas import tpu_sc as plsc

_N = 10000
_NP = 10240
_E = 160000
_EP = 163840
_NG = 64
_RATIO = 0.8
_HID = 128
_CH = 128
_NSUB = 16
_NW = 2 * _NSUB
_EPW = _EP // _NW
_CPT = _EPW // _CH
_RPT = _NP // _NSUB


def _sc_segsum(xn, src, dst):
  mesh = plsc.VectorSubcoreMesh(core_axis_name="c", subcore_axis_name="s")

  def body(xn_hbm, src_hbm, dst_hbm, out_hbm, src0, dst0, rows0, src1, dst1,
           rows1, agg_sh, sem0, sem1):
    cid = lax.axis_index("c")
    sid = lax.axis_index("s")
    wid = cid * _NSUB + sid
    base = wid * _EPW

    zero = jnp.zeros((16,), jnp.float32)

    def zbody(i, _):
      r = i // (_HID // 16)
      c = i % (_HID // 16)
      rows0[r, pl.ds(c * 16, 16)] = zero
      return 0

    lax.fori_loop(0, _CH * (_HID // 16), zbody, 0)
    for j in range(_RPT // _CH):
      pltpu.sync_copy(rows0, agg_sh.at[pl.ds(sid * _RPT + j * _CH, _CH)])

    pltpu.sync_copy(src_hbm.at[pl.ds(base, _CH)], src0)
    pltpu.sync_copy(dst_hbm.at[pl.ds(base, _CH)], dst0)
    pltpu.async_copy(xn_hbm.at[src0], rows0, sem0)
    plsc.subcore_barrier()

    def pair(j, _):
      c1 = 2 * j + 1
      c2 = 2 * j + 2

      @pl.when(c1 < _CPT)
      def _():
        pltpu.sync_copy(src_hbm.at[pl.ds(base + c1 * _CH, _CH)], src1)
        pltpu.sync_copy(dst_hbm.at[pl.ds(base + c1 * _CH, _CH)], dst1)
        pltpu.async_copy(xn_hbm.at[src1], rows1, sem1)

      pltpu.make_async_copy(xn_hbm.at[src0], rows0, sem0).wait()
      pltpu.sync_copy(rows0, agg_sh.at[dst0], add=True)

      @pl.when(c2 < _CPT)
      def _():
        pltpu.sync_copy(src_hbm.at[pl.ds(base + c2 * _CH, _CH)], src0)
        pltpu.sync_copy(dst_hbm.at[pl.ds(base + c2 * _CH, _CH)], dst0)
        pltpu.async_copy(xn_hbm.at[src0], rows0, sem0)

      @pl.when(c1 < _CPT)
      def _():
        pltpu.make_async_copy(xn_hbm.at[src1], rows1, sem1).wait()
        pltpu.sync_copy(rows1, agg_sh.at[dst1], add=True)

      return 0

    lax.fori_loop(0, (_CPT + 1) // 2, pair, 0)

    plsc.subcore_barrier()
    pltpu.sync_copy(agg_sh.at[pl.ds(sid * _RPT, _RPT)],
                    out_hbm.at[cid, pl.ds(sid * _RPT, _RPT)])

  return pl.kernel(
      body,
      out_type=jax.ShapeDtypeStruct((2, _NP, _HID), jnp.float32),
      mesh=mesh,
      scratch_types=[
          pltpu.VMEM((_CH,), jnp.int32),
          pltpu.VMEM((_CH,), jnp.int32),
          pltpu.VMEM((_CH, _HID), jnp.float32),
          pltpu.VMEM((_CH,), jnp.int32),
          pltpu.VMEM((_CH,), jnp.int32),
          pltpu.VMEM((_CH, _HID), jnp.float32),
          pltpu.VMEM_SHARED((_NP, _HID), jnp.float32),
          pltpu.SemaphoreType.DMA,
          pltpu.SemaphoreType.DMA,
      ],
  )(xn, src, dst)


def _conv(hprev, parts, Wr, Wn, b, p, prow):
  K = hprev.shape[1]
  MB = 1024
  npart = len(parts)

  def body(*refs):
    h_ref, s_ref = refs[-2], refs[-1]
    hp_ref = refs[0]
    part_refs = refs[1:1 + npart]
    wr_ref, wn_ref, b_ref, p_ref = refs[1 + npart:1 + npart + 4]
    prow_ref = refs[1 + npart + 4]
    if npart == 2:
      agg = part_refs[0][...] + part_refs[1][...]
    else:
      agg = jnp.concatenate(
          [part_refs[0][...] + part_refs[1][...],
           part_refs[2][...] + part_refs[3][...]], axis=1)
    h = jnp.maximum(
        jnp.dot(agg, wn_ref[...], preferred_element_type=jnp.float32) +
        jnp.dot(hp_ref[...], wr_ref[...], preferred_element_type=jnp.float32)
        + b_ref[...], 0.0)
    h_ref[...] = h
    pr = prow_ref[...]
    nrm = jnp.sqrt(jnp.sum(pr * pr)) + 1e-16
    u = jnp.dot(h, p_ref[...], preferred_element_type=jnp.float32)
    s_ref[...] = jnp.tanh(u / nrm)

  return pl.pallas_call(
      body,
      grid=(_NP // MB,),
      in_specs=[pl.BlockSpec((MB, K), lambda i: (i, 0))] +
               [pl.BlockSpec((MB, _HID), lambda i: (i, 0))] * npart +
               [
                   pl.BlockSpec((K, _HID), lambda i: (0, 0)),
                   pl.BlockSpec((K, _HID), lambda i: (0, 0)),
                   pl.BlockSpec((1, _HID), lambda i: (0, 0)),
                   pl.BlockSpec((_HID, 1), lambda i: (0, 0)),
                   pl.BlockSpec((1, _HID), lambda i: (0, 0)),
               ],
      out_specs=[
          pl.BlockSpec((MB, _HID), lambda i: (i, 0)),
          pl.BlockSpec((MB, 1), lambda i: (i, 0)),
      ],
      out_shape=[
          jax.ShapeDtypeStruct((_NP, _HID), jnp.float32),
          jax.ShapeDtypeStruct((_NP, 1), jnp.float32),
      ],
  )(hprev, *parts, Wr, Wn, b, p, prow)


def _rank(s_c, bc_c, bo_c, t_c, s_r, bc_r, bo_r, t_r):
  MB = 1024
  NB = _NP // MB

  def body(si_ref, bci_ref, boi_ref, ti_ref, sj_ref, bcj_ref, boj_ref,
           tj_ref, rank_ref, cnt_ref, cntt_ref):
    i = pl.program_id(0)
    j = pl.program_id(1)

    @pl.when(j == 0)
    def _():
      rank_ref[...] = jnp.zeros_like(rank_ref)
      bci = bci_ref[...]
      gi = lax.broadcasted_iota(jnp.int32, (1, 128), 1)
      eq = (bci == gi).astype(jnp.int32)

      @pl.when(i == 0)
      def _():
        cnt_ref[...] = jnp.zeros_like(cnt_ref)

      cnt_ref[...] += jnp.sum(eq, axis=0, keepdims=True)

    @pl.when((i == 0) & (j == 0))
    def _():
      cntt_ref[...] = jnp.zeros_like(cntt_ref)

    @pl.when(i == 0)
    def _():
      bcj = bcj_ref[...]
      gc = lax.broadcasted_iota(jnp.int32, (128, 1), 0)
      eqt = (bcj == gc).astype(jnp.int32)
      cntt_ref[...] += jnp.sum(eqt, axis=1, keepdims=True)

    lo_i = boi_ref[0, 0]
    hi_i = boi_ref[MB - 1, 0]
    lo_j = boj_ref[0, 0]
    hi_j = boj_ref[0, MB - 1]

    @pl.when((lo_j <= hi_i) & (hi_j >= lo_i))
    def _():
      si = si_ref[...]
      ti = ti_ref[...]
      bi = bci_ref[...]
      sj = sj_ref[...]
      tj = tj_ref[...]
      bj = bcj_ref[...]
      better = (sj > si) | ((sj == si) & (tj < ti))
      m = (better & (bj == bi)).astype(jnp.int32)
      rank_ref[...] += jnp.sum(m, axis=1, keepdims=True)

  return pl.pallas_call(
      body,
      grid=(NB, NB),
      in_specs=[
          pl.BlockSpec((MB, 1), lambda i, j: (i, 0)),
          pl.BlockSpec((MB, 1), lambda i, j: (i, 0)),
          pl.BlockSpec((MB, 1), lambda i, j: (i, 0)),
          pl.BlockSpec((MB, 1), lambda i, j: (i, 0)),
          pl.BlockSpec((1, MB), lambda i, j: (0, j)),
          pl.BlockSpec((1, MB), lambda i, j: (0, j)),
          pl.BlockSpec((1, MB), lambda i, j: (0, j)),
          pl.BlockSpec((1, MB), lambda i, j: (0, j)),
      ],
      out_specs=[
          pl.BlockSpec((MB, 1), lambda i, j: (i, 0)),
          pl.BlockSpec((1, 128), lambda i, j: (0, 0)),
          pl.BlockSpec((128, 1), lambda i, j: (0, 0)),
      ],
      out_shape=[
          jax.ShapeDtypeStruct((_NP, 1), jnp.int32),
          jax.ShapeDtypeStruct((1, 128), jnp.int32),
          jax.ShapeDtypeStruct((128, 1), jnp.int32),
      ],
  )(s_c, bc_c, bo_c, t_c, s_r, bc_r, bo_r, t_r)


def _apply(h, s_c, rank_c, cnt, bc_c, bo_c):
  MB = 1024
  NB = _NP // MB

  def body(h_ref, s_ref, r_ref, cnt_ref, bc_ref, bo_ref, hn_ref, bn_ref,
           gmax_ref, gsum_ref):
    pid = pl.program_id(0)

    @pl.when(pid == 0)
    def _():
      gmax_ref[...] = jnp.full_like(gmax_ref[...], -jnp.inf)
      gsum_ref[...] = jnp.zeros_like(gsum_ref[...])

    cntv = cnt_ref[...]
    kv = jnp.ceil(cntv.astype(jnp.float32) * _RATIO).astype(jnp.int32)
    bc = bc_ref[...]
    gi = lax.broadcasted_iota(jnp.int32, (1, 128), 1)
    kn = jnp.sum(jnp.where(bc == gi, kv, 0), axis=1, keepdims=True)
    rk = r_ref[...]
    sv = s_ref[...]
    keep = (rk < kn) & (bc < _NG)
    hn = jnp.where(keep, h_ref[...] * sv, 0.0)
    hn_ref[...] = hn
    bn = jnp.where(keep, bc, _NG)
    bn_ref[...] = bn

    glo = bo_ref[0, 0]
    ghi = jnp.minimum(bo_ref[MB - 1, 0], _NG - 1)

    def gbody(g, _):
      m = bn == g
      vals = jnp.where(m, hn, -jnp.inf)
      cur = gmax_ref[pl.ds(g, 1), :]
      gmax_ref[pl.ds(g, 1), :] = jnp.maximum(
          cur, jnp.max(vals, axis=0, keepdims=True))
      gsum_ref[pl.ds(g, 1), :] += jnp.sum(
          jnp.where(m, hn, 0.0), axis=0, keepdims=True)
      return 0

    lax.fori_loop(glo, ghi + 1, gbody, 0)

  return pl.pallas_call(
      body,
      grid=(NB,),
      in_specs=[
          pl.BlockSpec((MB, _HID), lambda i: (i, 0)),
          pl.BlockSpec((MB, 1), lambda i: (i, 0)),
          pl.BlockSpec((MB, 1), lambda i: (i, 0)),
          pl.BlockSpec((1, 128), lambda i: (0, 0)),
          pl.BlockSpec((MB, 1), lambda i: (i, 0)),
          pl.BlockSpec((MB, 1), lambda i: (i, 0)),
      ],
      out_specs=[
          pl.BlockSpec((MB, _HID), lambda i: (i, 0)),
          pl.BlockSpec((MB, 1), lambda i: (i, 0)),
          pl.BlockSpec((_NG, 128), lambda i: (0, 0)),
          pl.BlockSpec((_NG, 128), lambda i: (0, 0)),
      ],
      out_shape=[
          jax.ShapeDtypeStruct((_NP, _HID), jnp.float32),
          jax.ShapeDtypeStruct((_NP, 1), jnp.int32),
          jax.ShapeDtypeStruct((_NG, 128), jnp.float32),
          jax.ShapeDtypeStruct((_NG, 128), jnp.float32),
      ],
  )(h, s_c, rank_c, cnt, bc_c, bo_c)


def _head(m1, s1, c1, m2, s2, c2, m3, s3, c3, w1a, w1b, b1, w2, b2, w3, b3):
  def body(m1_ref, s1_ref, c1_ref, m2_ref, s2_ref, c2_ref, m3_ref, s3_ref,
           c3_ref, w1a_ref, w1b_ref, b1_ref, w2_ref, b2_ref, w3_ref, b3_ref,
           out_ref):

    def rd(mref, sref, cref):
      cv = cref[...]
      kcol = jnp.ceil(cv.astype(jnp.float32) * _RATIO)[:_NG, :]
      mx = mref[...]
      mx = jnp.where(jnp.isfinite(mx), mx, 0.0)
      mean = sref[...] / jnp.maximum(kcol, 1.0)
      return mx, mean

    mA, eA = rd(m1_ref, s1_ref, c1_ref)
    mB, eB = rd(m2_ref, s2_ref, c2_ref)
    mC, eC = rd(m3_ref, s3_ref, c3_ref)
    zmax = mA + mB + mC
    zmean = eA + eB + eC
    hh = jnp.maximum(
        jnp.dot(zmax, w1a_ref[...], preferred_element_type=jnp.float32) +
        jnp.dot(zmean, w1b_ref[...], preferred_element_type=jnp.float32) +
        b1_ref[...], 0.0)
    hh = jnp.maximum(
        jnp.dot(hh, w2_ref[...], preferred_element_type=jnp.float32) +
        b2_ref[...], 0.0)
    o = jnp.dot(hh, w3_ref[...], preferred_element_type=jnp.float32) + b3_ref[...]
    o = o - jnp.max(o, axis=1, keepdims=True)
    out_ref[...] = o - jnp.log(jnp.sum(jnp.exp(o), axis=1, keepdims=True))

  return pl.pallas_call(
      body,
      out_shape=jax.ShapeDtypeStruct((_NG, 16), jnp.float32),
  )(m1, s1, c1, m2, s2, c2, m3, s3, c3, w1a, w1b, b1, w2, b2, w3, b3)


def kernel(x, edge_index, batch, W1r, W1n, b1, p1, W2r, W2n, b2, p2, W3r, W3n,
           b3, p3, L1w, L1b, L2w, L2b, L3w, L3b):
  xp = jnp.zeros((_NP, x.shape[1]), jnp.float32).at[:_N].set(x)
  xA = xp[:, :_HID]
  xB = xp[:, _HID:]
  pad_e = jnp.full((_EP - _E,), _N, jnp.int32)
  srcp = jnp.concatenate([edge_index[0], pad_e])
  dstp = jnp.concatenate([edge_index[1], pad_e])
  bor = jnp.full((_NP,), _NG, jnp.int32).at[:_N].set(batch)
  bor_c = bor.reshape(_NP, 1)
  bor_r = bor.reshape(1, _NP)
  bc_c, bc_r = bor_c, bor_r
  t_c = jnp.arange(_NP, dtype=jnp.int32).reshape(_NP, 1)
  t_r = t_c.reshape(1, _NP)
  hprev = xp

  reads = []
  for li, (Wr, Wn, bb, pp) in enumerate(((W1r, W1n, b1, p1), (W2r, W2n, b2, p2),
                                         (W3r, W3n, b3, p3))):
    if li == 0:
      pa = _sc_segsum(xA, srcp, dstp)
      pb = _sc_segsum(xB, srcp, dstp)
      parts = (pa[0], pa[1], pb[0], pb[1])
    else:
      pa = _sc_segsum(hprev, srcp, dstp)
      parts = (pa[0], pa[1])
    h, s_c = _conv(hprev, parts, Wr, Wn, bb.reshape(1, _HID),
                   pp.reshape(_HID, 1), pp.reshape(1, _HID))
    s_r = s_c.reshape(1, _NP)
    rank_c, cnt, cntt = _rank(s_c, bc_c, bor_c, t_c, s_r, bc_r, bor_r, t_r)
    hn, bn_c, gmax, gsum = _apply(h, s_c, rank_c, cnt, bc_c, bor_c)
    reads.append((gmax, gsum, cntt))
    hprev = hn
    bc_c, bc_r = bn_c, bn_c.reshape(1, _NP)
    t_c, t_r = rank_c, rank_c.reshape(1, _NP)

  (m1, s1, c1), (m2, s2, c2), (m3, s3, c3) = reads
  return _head(m1, s1, c1, m2, s2, c2, m3, s3, c3, L1w[:_HID], L1w[_HID:],
               L1b.reshape(1, _HID), L2w, L2b.reshape(1, 64), L3w,
               L3b.reshape(1, 16))
